# baseline (speedup 1.0000x reference)
import jax
import jax.numpy as jnp
from jax import lax
from jax.experimental import pallas as pl
from jax.experimental.pallas import tpu as pltpu
from jax.experimental.pallas import tpu_sc as plsc

N_NODES = 10000
D = 128
NC = 2
NS = 16
NW = NC * NS
CHUNK = 128

E = 320000
NCHUNKS = -(-E // (NW * CHUNK))
NCHUNKS = NCHUNKS + (NCHUNKS % 2)
EPW = NCHUNKS * CHUNK
E_PAD = EPW * NW
GROUP = 8

ZPT = 632
N_PAD = ZPT * NS
OPT = 624
OREM = N_NODES - OPT * NS


def _mm_body(x_ref, w_ref, b_ref, o_ref):
    o_ref[...] = (
        jnp.dot(x_ref[...], w_ref[...], preferred_element_type=jnp.float32)
        + b_ref[...]
    )


def _comb_body(p_ref, b_ref, o_ref):
    o_ref[...] = p_ref[0] + p_ref[1] + b_ref[...]


def _sc_body(h_hbm, src_flat, dst_flat, out_hbm,
             srcv0, srcv1, dstv0, dstv1, rows_a, rows_b, acc, sem):
    cid = lax.axis_index("c")
    sid = lax.axis_index("s")
    wid = cid * NS + sid

    z16 = jnp.zeros((16,), jnp.float32)

    def _zero_row(r, carry):
        for j in range(D // 16):
            rows_a[r, pl.ds(16 * j, 16)] = z16
        return carry

    lax.fori_loop(0, CHUNK, _zero_row, 0)

    zbase = pl.multiple_of(sid * ZPT, 8)
    for k in range(ZPT // CHUNK):
        pltpu.sync_copy(rows_a.at[pl.ds(0, CHUNK)],
                        acc.at[pl.ds(zbase + k * CHUNK, CHUNK)])
    zrem = ZPT % CHUNK
    if zrem:
        pltpu.sync_copy(rows_a.at[pl.ds(0, zrem)],
                        acc.at[pl.ds(zbase + (ZPT // CHUNK) * CHUNK, zrem)])

    plsc.subcore_barrier()

    base0 = wid * EPW

    def _chunk(j, carry):
        base = pl.multiple_of(base0 + j * CHUNK, CHUNK)
        pltpu.sync_copy(src_flat.at[pl.ds(base, CHUNK)], srcv0)
        pltpu.sync_copy(dst_flat.at[pl.ds(base, CHUNK)], dstv0)
        pltpu.async_copy(h_hbm.at[srcv0], rows_a, sem).wait()
        pltpu.sync_copy(rows_a, acc.at[dstv0], add=True)
        return carry

    lax.fori_loop(0, NCHUNKS, _chunk, 0)

    plsc.subcore_barrier()

    obase = pl.multiple_of(sid * OPT, 8)
    for k in range(OPT // CHUNK):
        pltpu.sync_copy(acc.at[pl.ds(obase + k * CHUNK, CHUNK)],
                        out_hbm.at[cid].at[pl.ds(obase + k * CHUNK, CHUNK)])
    orem = OPT % CHUNK
    if orem:
        pltpu.sync_copy(acc.at[pl.ds(obase + (OPT // CHUNK) * CHUNK, orem)],
                        out_hbm.at[cid].at[pl.ds(obase + (OPT // CHUNK) * CHUNK, orem)])

    @pl.when(sid == NS - 1)
    def _tail():
        pltpu.sync_copy(acc.at[pl.ds(OPT * NS, OREM)],
                        out_hbm.at[cid].at[pl.ds(OPT * NS, OREM)])


def kernel(x, edge_index, W, b_dense, bias):
    src = edge_index[0].astype(jnp.int32)
    dst = edge_index[1].astype(jnp.int32)
    pad = E_PAD - E
    src = jnp.concatenate([src, jnp.zeros((pad,), jnp.int32)])
    dst = jnp.concatenate([dst, jnp.full((pad,), N_NODES, jnp.int32)])

    b2 = b_dense[None, :]
    h = pl.pallas_call(
        _mm_body,
        grid=(10,),
        in_specs=[
            pl.BlockSpec((N_NODES // 10, D), lambda i: (i, 0)),
            pl.BlockSpec((D, D), lambda i: (0, 0)),
            pl.BlockSpec((1, D), lambda i: (0, 0)),
        ],
        out_specs=pl.BlockSpec((N_NODES // 10, D), lambda i: (i, 0)),
        out_shape=jax.ShapeDtypeStruct((N_NODES, D), jnp.float32),
    )(x, W, b2)

    sc_fn = pl.kernel(
        _sc_body,
        out_type=jax.ShapeDtypeStruct((NC, N_NODES, D), jnp.float32),
        mesh=plsc.VectorSubcoreMesh(core_axis_name="c", subcore_axis_name="s"),
        scratch_types=[
            pltpu.VMEM((CHUNK,), jnp.int32),
            pltpu.VMEM((CHUNK,), jnp.int32),
            pltpu.VMEM((CHUNK,), jnp.int32),
            pltpu.VMEM((CHUNK,), jnp.int32),
            pltpu.VMEM((CHUNK, D), jnp.float32),
            pltpu.VMEM((CHUNK, D), jnp.float32),
            pltpu.VMEM_SHARED((N_PAD, D), jnp.float32),
            pltpu.SemaphoreType.DMA,
        ],
    )
    partials = sc_fn(h, src, dst)

    bias2 = bias[None, :]
    out = pl.pallas_call(
        _comb_body,
        grid=(10,),
        in_specs=[
            pl.BlockSpec((NC, N_NODES // 10, D), lambda i: (0, i, 0)),
            pl.BlockSpec((1, D), lambda i: (0, 0)),
        ],
        out_specs=pl.BlockSpec((N_NODES // 10, D), lambda i: (i, 0)),
        out_shape=jax.ShapeDtypeStruct((N_NODES, D), jnp.float32),
    )(partials, bias2)
    return out


# speedup vs baseline: 2.0801x; 2.0801x over previous
import jax
import jax.numpy as jnp
from jax import lax
from jax.experimental import pallas as pl
from jax.experimental.pallas import tpu as pltpu
from jax.experimental.pallas import tpu_sc as plsc

N_NODES = 10000
D = 128
DH = D // 2
NC = 2
NS = 16
CHUNK = 128

E = 320000
TCH = 160
NCHUNKS = TCH * NS
E_PAD = NCHUNKS * CHUNK
HALF = TCH // 2

ZPT = 632
N_PAD = ZPT * NS
OPT = 624
OREM = N_NODES - OPT * NS


def _mm_body(x_ref, w_ref, b_ref, o_ref):
    o_ref[...] = (
        jnp.dot(x_ref[...], w_ref[0], preferred_element_type=jnp.float32)
        + b_ref[0]
    )


def _comb_body(p_ref, b_ref, o_ref):
    o_ref[...] = jnp.concatenate([p_ref[0], p_ref[1]], axis=1) + b_ref[...]


def _sc_body(hflat_hbm, idx_hbm, out_hbm, srcv, dstv, rows_v, idxs, hsh, acc, sem):
    cid = lax.axis_index("c")
    sid = lax.axis_index("s")

    z16 = jnp.zeros((16,), jnp.float32)

    def _zero_row(r, carry):
        for j in range(DH // 16):
            rows_v[r, pl.ds(16 * j, 16)] = z16
        return carry

    lax.fori_loop(0, CHUNK, _zero_row, 0)

    zbase = pl.multiple_of(sid * ZPT, 8)
    for k in range(ZPT // CHUNK):
        pltpu.sync_copy(rows_v.at[pl.ds(0, CHUNK)],
                        acc.at[pl.ds(zbase + k * CHUNK, CHUNK)])
    zrem = ZPT % CHUNK
    if zrem:
        pltpu.sync_copy(rows_v.at[pl.ds(0, zrem)],
                        acc.at[pl.ds(zbase + (ZPT // CHUNK) * CHUNK, zrem)])

    hbase = pl.multiple_of(sid * OPT, 8)
    cbase = pl.multiple_of(cid * N_NODES, 8)
    pltpu.sync_copy(hflat_hbm.at[pl.ds(cbase + hbase, OPT)], hsh.at[pl.ds(hbase, OPT)])

    @pl.when(sid == NS - 1)
    def _stage_tail():
        pltpu.sync_copy(hflat_hbm.at[pl.ds(cbase + OPT * NS, OREM)],
                        hsh.at[pl.ds(OPT * NS, OREM)])

    plsc.subcore_barrier()

    for half in range(2):
        gsrc = pl.multiple_of(sid * TCH + half * HALF, 8)
        gdst = pl.multiple_of(sid * HALF, 8)
        pltpu.sync_copy(idx_hbm.at[pl.ds(gsrc, HALF)], idxs.at[pl.ds(gdst, HALF)])

        def _chunk(j, carry):
            r = sid * HALF + j
            pltpu.sync_copy(idxs.at[r].at[0], srcv)
            pltpu.sync_copy(idxs.at[r].at[1], dstv)
            pltpu.async_copy(hsh.at[srcv], rows_v, sem).wait()
            pltpu.sync_copy(rows_v, acc.at[dstv], add=True)
            return carry

        lax.fori_loop(0, HALF, _chunk, 0)

    plsc.subcore_barrier()

    obase = pl.multiple_of(sid * OPT, 8)
    for k in range(OPT // CHUNK):
        pltpu.sync_copy(acc.at[pl.ds(obase + k * CHUNK, CHUNK)],
                        out_hbm.at[cid].at[pl.ds(obase + k * CHUNK, CHUNK)])
    orem = OPT % CHUNK
    if orem:
        pltpu.sync_copy(acc.at[pl.ds(obase + (OPT // CHUNK) * CHUNK, orem)],
                        out_hbm.at[cid].at[pl.ds(obase + (OPT // CHUNK) * CHUNK, orem)])

    @pl.when(sid == NS - 1)
    def _tail():
        pltpu.sync_copy(acc.at[pl.ds(OPT * NS, OREM)],
                        out_hbm.at[cid].at[pl.ds(OPT * NS, OREM)])


def kernel(x, edge_index, W, b_dense, bias):
    src = edge_index[0].astype(jnp.int32)
    dst = edge_index[1].astype(jnp.int32)
    pad = E_PAD - E
    src = jnp.concatenate([src, jnp.zeros((pad,), jnp.int32)])
    dst = jnp.concatenate([dst, jnp.full((pad,), N_NODES, jnp.int32)])
    src2 = src.reshape(NCHUNKS, CHUNK)
    dst2 = dst.reshape(NCHUNKS, CHUNK)
    idx2 = jnp.stack([src2, dst2], axis=1)

    b2 = b_dense.reshape(NC, 1, DH)
    W2 = jnp.stack([W[:, :DH], W[:, DH:]], axis=0)
    hflat = pl.pallas_call(
        _mm_body,
        grid=(10, NC),
        in_specs=[
            pl.BlockSpec((N_NODES // 10, D), lambda i, c: (i, 0)),
            pl.BlockSpec((1, D, DH), lambda i, c: (c, 0, 0)),
            pl.BlockSpec((1, 1, DH), lambda i, c: (c, 0, 0)),
        ],
        out_specs=pl.BlockSpec((N_NODES // 10, DH), lambda i, c: (c * 10 + i, 0)),
        out_shape=jax.ShapeDtypeStruct((NC * N_NODES, DH), jnp.float32),
    )(x, W2, b2)

    sc_fn = pl.kernel(
        _sc_body,
        out_type=jax.ShapeDtypeStruct((NC, N_NODES, DH), jnp.float32),
        mesh=plsc.VectorSubcoreMesh(core_axis_name="c", subcore_axis_name="s"),
        compiler_params=pltpu.CompilerParams(use_tc_tiling_on_sc=False),
        scratch_types=[
            pltpu.VMEM((CHUNK,), jnp.int32),
            pltpu.VMEM((CHUNK,), jnp.int32),
            pltpu.VMEM((CHUNK, DH), jnp.float32),
            pltpu.VMEM_SHARED((NS * HALF, 2, CHUNK), jnp.int32),
            pltpu.VMEM_SHARED((N_NODES, DH), jnp.float32),
            pltpu.VMEM_SHARED((N_PAD, DH), jnp.float32),
            pltpu.SemaphoreType.DMA,
        ],
    )
    partials = sc_fn(hflat, idx2)

    bias2 = bias[None, :]
    out = pl.pallas_call(
        _comb_body,
        grid=(10,),
        in_specs=[
            pl.BlockSpec((NC, N_NODES // 10, DH), lambda i: (0, i, 0)),
            pl.BlockSpec((1, D), lambda i: (0, 0)),
        ],
        out_specs=pl.BlockSpec((N_NODES // 10, D), lambda i: (i, 0)),
        out_shape=jax.ShapeDtypeStruct((N_NODES, D), jnp.float32),
    )(partials, bias2)
    return out


# speedup vs baseline: 2.5589x; 1.2302x over previous
import jax
import jax.numpy as jnp
from jax import lax
from jax.experimental import pallas as pl
from jax.experimental.pallas import tpu as pltpu
from jax.experimental.pallas import tpu_sc as plsc

N_NODES = 10000
D = 128
DH = D // 2
NC = 2
NS = 16
CHUNK = 128

E = 320000
TCH = 160
NCHUNKS = TCH * NS
E_PAD = NCHUNKS * CHUNK
HALF = TCH // 2

ZPT = 632
N_PAD = ZPT * NS
OPT = 624
OREM = N_NODES - OPT * NS


def _mm_body(x_ref, w_ref, b_ref, o_ref):
    o_ref[...] = (
        jnp.dot(x_ref[...], w_ref[0], preferred_element_type=jnp.float32)
        + b_ref[0]
    )


def _comb_body(p_ref, b_ref, o_ref):
    o_ref[...] = jnp.concatenate([p_ref[0], p_ref[1]], axis=1) + b_ref[...]


def _sc_body(hflat_hbm, idx_hbm, out_hbm,
             srcv0, srcv1, dstv0, dstv1, rows_a, rows_b, idxs, hsh, acc, sem):
    cid = lax.axis_index("c")
    sid = lax.axis_index("s")

    z16 = jnp.zeros((16,), jnp.float32)

    def _zero_row(r, carry):
        for j in range(DH // 16):
            rows_a[r, pl.ds(16 * j, 16)] = z16
        return carry

    lax.fori_loop(0, CHUNK, _zero_row, 0)

    zbase = pl.multiple_of(sid * ZPT, 8)
    for k in range(ZPT // CHUNK):
        pltpu.sync_copy(rows_a.at[pl.ds(0, CHUNK)],
                        acc.at[pl.ds(zbase + k * CHUNK, CHUNK)])
    zrem = ZPT % CHUNK
    if zrem:
        pltpu.sync_copy(rows_a.at[pl.ds(0, zrem)],
                        acc.at[pl.ds(zbase + (ZPT // CHUNK) * CHUNK, zrem)])

    hbase = pl.multiple_of(sid * OPT, 8)
    cbase = pl.multiple_of(cid * N_NODES, 8)
    pltpu.sync_copy(hflat_hbm.at[pl.ds(cbase + hbase, OPT)], hsh.at[pl.ds(hbase, OPT)])

    @pl.when(sid == NS - 1)
    def _stage_tail():
        pltpu.sync_copy(hflat_hbm.at[pl.ds(cbase + OPT * NS, OREM)],
                        hsh.at[pl.ds(OPT * NS, OREM)])

    plsc.subcore_barrier()

    srcv = (srcv0, srcv1)
    dstv = (dstv0, dstv1)
    rows = (rows_a, rows_b)

    for half in range(2):
        gsrc = pl.multiple_of(sid * TCH + half * HALF, 8)
        gdst = pl.multiple_of(sid * HALF, 8)
        pltpu.sync_copy(idx_hbm.at[pl.ds(gsrc, HALF)], idxs.at[pl.ds(gdst, HALF)])

        r0 = sid * HALF
        pltpu.sync_copy(idxs.at[r0].at[0], srcv0)
        pltpu.sync_copy(idxs.at[r0].at[1], dstv0)
        pltpu.async_copy(hsh.at[srcv0], rows_a, sem)

        def _pair(jj, carry):
            for s in range(2):
                j = 2 * jj + s
                p, q = s, 1 - s
                r = sid * HALF + j
                pltpu.make_async_copy(hsh.at[srcv[p]], rows[p], sem).wait()
                if s == 0:
                    nxt = True
                else:
                    nxt = jj < HALF // 2 - 1

                def _prefetch():
                    pltpu.sync_copy(idxs.at[r + 1].at[0], srcv[q])
                    pltpu.sync_copy(idxs.at[r + 1].at[1], dstv[q])
                    pltpu.async_copy(hsh.at[srcv[q]], rows[q], sem)

                if nxt is True:
                    _prefetch()
                else:
                    pl.when(nxt)(_prefetch)
                pltpu.sync_copy(rows[p], acc.at[dstv[p]], add=True)
            return carry

        lax.fori_loop(0, HALF // 2, _pair, 0)

    plsc.subcore_barrier()

    obase = pl.multiple_of(sid * OPT, 8)
    for k in range(OPT // CHUNK):
        pltpu.sync_copy(acc.at[pl.ds(obase + k * CHUNK, CHUNK)],
                        out_hbm.at[cid].at[pl.ds(obase + k * CHUNK, CHUNK)])
    orem = OPT % CHUNK
    if orem:
        pltpu.sync_copy(acc.at[pl.ds(obase + (OPT // CHUNK) * CHUNK, orem)],
                        out_hbm.at[cid].at[pl.ds(obase + (OPT // CHUNK) * CHUNK, orem)])

    @pl.when(sid == NS - 1)
    def _tail():
        pltpu.sync_copy(acc.at[pl.ds(OPT * NS, OREM)],
                        out_hbm.at[cid].at[pl.ds(OPT * NS, OREM)])


def kernel(x, edge_index, W, b_dense, bias):
    src = edge_index[0].astype(jnp.int32)
    dst = edge_index[1].astype(jnp.int32)
    pad = E_PAD - E
    src = jnp.concatenate([src, jnp.zeros((pad,), jnp.int32)])
    dst = jnp.concatenate([dst, jnp.full((pad,), N_NODES, jnp.int32)])
    src2 = src.reshape(NCHUNKS, CHUNK)
    dst2 = dst.reshape(NCHUNKS, CHUNK)
    idx2 = jnp.stack([src2, dst2], axis=1)

    b2 = b_dense.reshape(NC, 1, DH)
    W2 = jnp.stack([W[:, :DH], W[:, DH:]], axis=0)
    hflat = pl.pallas_call(
        _mm_body,
        grid=(10, NC),
        in_specs=[
            pl.BlockSpec((N_NODES // 10, D), lambda i, c: (i, 0)),
            pl.BlockSpec((1, D, DH), lambda i, c: (c, 0, 0)),
            pl.BlockSpec((1, 1, DH), lambda i, c: (c, 0, 0)),
        ],
        out_specs=pl.BlockSpec((N_NODES // 10, DH), lambda i, c: (c * 10 + i, 0)),
        out_shape=jax.ShapeDtypeStruct((NC * N_NODES, DH), jnp.float32),
    )(x, W2, b2)

    sc_fn = pl.kernel(
        _sc_body,
        out_type=jax.ShapeDtypeStruct((NC, N_NODES, DH), jnp.float32),
        mesh=plsc.VectorSubcoreMesh(core_axis_name="c", subcore_axis_name="s"),
        compiler_params=pltpu.CompilerParams(use_tc_tiling_on_sc=False),
        scratch_types=[
            pltpu.VMEM((CHUNK,), jnp.int32),
            pltpu.VMEM((CHUNK,), jnp.int32),
            pltpu.VMEM((CHUNK,), jnp.int32),
            pltpu.VMEM((CHUNK,), jnp.int32),
            pltpu.VMEM((CHUNK, DH), jnp.float32),
            pltpu.VMEM((CHUNK, DH), jnp.float32),
            pltpu.VMEM_SHARED((NS * HALF, 2, CHUNK), jnp.int32),
            pltpu.VMEM_SHARED((N_NODES, DH), jnp.float32),
            pltpu.VMEM_SHARED((N_PAD, DH), jnp.float32),
            pltpu.SemaphoreType.DMA,
        ],
    )
    partials = sc_fn(hflat, idx2)

    bias2 = bias[None, :]
    out = pl.pallas_call(
        _comb_body,
        grid=(10,),
        in_specs=[
            pl.BlockSpec((NC, N_NODES // 10, DH), lambda i: (0, i, 0)),
            pl.BlockSpec((1, D), lambda i: (0, 0)),
        ],
        out_specs=pl.BlockSpec((N_NODES // 10, D), lambda i: (i, 0)),
        out_shape=jax.ShapeDtypeStruct((N_NODES, D), jnp.float32),
    )(partials, bias2)
    return out


# speedup vs baseline: 2.7548x; 1.0766x over previous
import jax
import jax.numpy as jnp
from jax import lax
from jax.experimental import pallas as pl
from jax.experimental.pallas import tpu as pltpu
from jax.experimental.pallas import tpu_sc as plsc

N_NODES = 10000
D = 128
DH = D // 2
NC = 2
NS = 16
CHUNK = 128

E = 320000
TCH = 160
NCHUNKS = TCH * NS
E_PAD = NCHUNKS * CHUNK
HALF = TCH // 2

ZPT = 632
N_PAD = ZPT * NS
OPT = 624
OREM = N_NODES - OPT * NS


def _mm_body(x_ref, w_ref, b_ref, o_ref):
    o_ref[...] = (
        jnp.dot(x_ref[...], w_ref[0], preferred_element_type=jnp.float32)
        + b_ref[0]
    )


def _sc_body(hflat_hbm, idx_hbm, bias_hbm, out_hbm,
             srcv0, srcv1, dstv0, dstv1, rows_a, rows_b, biasv,
             idxs, hsh, acc, sem):
    cid = lax.axis_index("c")
    sid = lax.axis_index("s")

    pltpu.sync_copy(bias_hbm.at[cid], biasv)

    def _bias_row(r, carry):
        for j in range(DH // 16):
            rows_a[r, pl.ds(16 * j, 16)] = biasv[pl.ds(16 * j, 16)]
        return carry

    lax.fori_loop(0, CHUNK, _bias_row, 0)

    zbase = pl.multiple_of(sid * ZPT, 8)
    for k in range(ZPT // CHUNK):
        pltpu.sync_copy(rows_a.at[pl.ds(0, CHUNK)],
                        acc.at[pl.ds(zbase + k * CHUNK, CHUNK)])
    zrem = ZPT % CHUNK
    if zrem:
        pltpu.sync_copy(rows_a.at[pl.ds(0, zrem)],
                        acc.at[pl.ds(zbase + (ZPT // CHUNK) * CHUNK, zrem)])

    hbase = pl.multiple_of(sid * OPT, 8)
    cbase = pl.multiple_of(cid * N_NODES, 8)
    pltpu.sync_copy(hflat_hbm.at[pl.ds(cbase + hbase, OPT)], hsh.at[pl.ds(hbase, OPT)])

    @pl.when(sid == NS - 1)
    def _stage_tail():
        pltpu.sync_copy(hflat_hbm.at[pl.ds(cbase + OPT * NS, OREM)],
                        hsh.at[pl.ds(OPT * NS, OREM)])

    plsc.subcore_barrier()

    srcv = (srcv0, srcv1)
    dstv = (dstv0, dstv1)
    rows = (rows_a, rows_b)

    for half in range(2):
        gsrc = pl.multiple_of(sid * TCH + half * HALF, 8)
        gdst = pl.multiple_of(sid * HALF, 8)
        pltpu.sync_copy(idx_hbm.at[pl.ds(gsrc, HALF)], idxs.at[pl.ds(gdst, HALF)])

        r0 = sid * HALF
        pltpu.sync_copy(idxs.at[r0].at[0], srcv0)
        pltpu.sync_copy(idxs.at[r0].at[1], dstv0)
        pltpu.async_copy(hsh.at[srcv0], rows_a, sem)

        def _pair(jj, carry):
            for s in range(2):
                j = 2 * jj + s
                p, q = s, 1 - s
                r = sid * HALF + j
                pltpu.make_async_copy(hsh.at[srcv[p]], rows[p], sem).wait()
                if s == 0:
                    nxt = True
                else:
                    nxt = jj < HALF // 2 - 1

                def _prefetch():
                    pltpu.sync_copy(idxs.at[r + 1].at[0], srcv[q])
                    pltpu.sync_copy(idxs.at[r + 1].at[1], dstv[q])
                    pltpu.async_copy(hsh.at[srcv[q]], rows[q], sem)

                if nxt is True:
                    _prefetch()
                else:
                    pl.when(nxt)(_prefetch)
                pltpu.sync_copy(rows[p], acc.at[dstv[p]], add=True)
            return carry

        lax.fori_loop(0, HALF // 2, _pair, 0)

    plsc.subcore_barrier()

    obase = pl.multiple_of(sid * OPT, 8)
    cob = pl.multiple_of(cid * DH, 8)
    for k in range(OPT // CHUNK):
        pltpu.sync_copy(acc.at[pl.ds(obase + k * CHUNK, CHUNK)],
                        out_hbm.at[pl.ds(obase + k * CHUNK, CHUNK), pl.ds(cob, DH)])
    orem = OPT % CHUNK
    if orem:
        pltpu.sync_copy(acc.at[pl.ds(obase + (OPT // CHUNK) * CHUNK, orem)],
                        out_hbm.at[pl.ds(obase + (OPT // CHUNK) * CHUNK, orem),
                                   pl.ds(cob, DH)])

    @pl.when(sid == NS - 1)
    def _tail():
        pltpu.sync_copy(acc.at[pl.ds(OPT * NS, OREM)],
                        out_hbm.at[pl.ds(OPT * NS, OREM), pl.ds(cob, DH)])


def kernel(x, edge_index, W, b_dense, bias):
    src = edge_index[0].astype(jnp.int32)
    dst = edge_index[1].astype(jnp.int32)
    pad = E_PAD - E
    src = jnp.concatenate([src, jnp.zeros((pad,), jnp.int32)])
    dst = jnp.concatenate([dst, jnp.full((pad,), N_NODES, jnp.int32)])
    src2 = src.reshape(NCHUNKS, CHUNK)
    dst2 = dst.reshape(NCHUNKS, CHUNK)
    idx2 = jnp.stack([src2, dst2], axis=1)

    b2 = b_dense.reshape(NC, 1, DH)
    W2 = jnp.stack([W[:, :DH], W[:, DH:]], axis=0)
    hflat = pl.pallas_call(
        _mm_body,
        grid=(10, NC),
        in_specs=[
            pl.BlockSpec((N_NODES // 10, D), lambda i, c: (i, 0)),
            pl.BlockSpec((1, D, DH), lambda i, c: (c, 0, 0)),
            pl.BlockSpec((1, 1, DH), lambda i, c: (c, 0, 0)),
        ],
        out_specs=pl.BlockSpec((N_NODES // 10, DH), lambda i, c: (c * 10 + i, 0)),
        out_shape=jax.ShapeDtypeStruct((NC * N_NODES, DH), jnp.float32),
    )(x, W2, b2)

    sc_fn = pl.kernel(
        _sc_body,
        out_type=jax.ShapeDtypeStruct((N_NODES, D), jnp.float32),
        mesh=plsc.VectorSubcoreMesh(core_axis_name="c", subcore_axis_name="s"),
        compiler_params=pltpu.CompilerParams(use_tc_tiling_on_sc=False),
        scratch_types=[
            pltpu.VMEM((CHUNK,), jnp.int32),
            pltpu.VMEM((CHUNK,), jnp.int32),
            pltpu.VMEM((CHUNK,), jnp.int32),
            pltpu.VMEM((CHUNK,), jnp.int32),
            pltpu.VMEM((CHUNK, DH), jnp.float32),
            pltpu.VMEM((CHUNK, DH), jnp.float32),
            pltpu.VMEM((DH,), jnp.float32),
            pltpu.VMEM_SHARED((NS * HALF, 2, CHUNK), jnp.int32),
            pltpu.VMEM_SHARED((N_NODES, DH), jnp.float32),
            pltpu.VMEM_SHARED((N_PAD, DH), jnp.float32),
            pltpu.SemaphoreType.DMA,
        ],
    )
    bias2 = bias.reshape(NC, DH)
    return sc_fn(hflat, idx2, bias2)


# speedup vs baseline: 2.8834x; 1.0467x over previous
import jax
import jax.numpy as jnp
from jax import lax
from jax.experimental import pallas as pl
from jax.experimental.pallas import tpu as pltpu
from jax.experimental.pallas import tpu_sc as plsc

N_NODES = 10000
D = 128
DH = D // 2
NC = 2
NS = 16
CHUNK = 128

E = 320000
TCH = 160
NCHUNKS = TCH * NS
E_PAD = NCHUNKS * CHUNK
HALF = TCH // 2
LCH = E // CHUNK
PADC = NCHUNKS - LCH
L15 = LCH - (NS - 1) * TCH - HALF

ZPT = 632
N_PAD = ZPT * NS
OPT = 624
OREM = N_NODES - OPT * NS


def _mm_body(x_ref, w_ref, b_ref, o_ref):
    o_ref[...] = (
        jnp.dot(x_ref[...], w_ref[0], preferred_element_type=jnp.float32)
        + b_ref[0]
    )


def _sc_body(hflat_hbm, eidx_hbm, pad_hbm, bias_hbm, out_hbm,
             srcv0, srcv1, dstv0, dstv1, rows_a, rows_b, biasv,
             idxs, hsh, acc, sem):
    cid = lax.axis_index("c")
    sid = lax.axis_index("s")

    pltpu.sync_copy(bias_hbm.at[cid], biasv)

    def _bias_row(r, carry):
        for j in range(DH // 16):
            rows_a[r, pl.ds(16 * j, 16)] = biasv[pl.ds(16 * j, 16)]
        return carry

    lax.fori_loop(0, CHUNK, _bias_row, 0)

    zbase = pl.multiple_of(sid * ZPT, 8)
    for k in range(ZPT // CHUNK):
        pltpu.sync_copy(rows_a.at[pl.ds(0, CHUNK)],
                        acc.at[pl.ds(zbase + k * CHUNK, CHUNK)])
    zrem = ZPT % CHUNK
    if zrem:
        pltpu.sync_copy(rows_a.at[pl.ds(0, zrem)],
                        acc.at[pl.ds(zbase + (ZPT // CHUNK) * CHUNK, zrem)])

    hbase = pl.multiple_of(sid * OPT, 8)
    cbase = pl.multiple_of(cid * N_NODES, 8)
    pltpu.sync_copy(hflat_hbm.at[pl.ds(cbase + hbase, OPT)], hsh.at[pl.ds(hbase, OPT)])

    @pl.when(sid == NS - 1)
    def _stage_tail():
        pltpu.sync_copy(hflat_hbm.at[pl.ds(cbase + OPT * NS, OREM)],
                        hsh.at[pl.ds(OPT * NS, OREM)])

    plsc.subcore_barrier()

    srcv = (srcv0, srcv1)
    dstv = (dstv0, dstv1)
    rows = (rows_a, rows_b)

    for half in range(2):
        eb = pl.multiple_of((sid * TCH + half * HALF) * CHUNK, 8)
        if half == 0:
            for rr in range(2):
                pltpu.sync_copy(eidx_hbm.at[rr].at[pl.ds(eb, HALF * CHUNK)],
                                idxs.at[rr].at[sid])
        else:
            @pl.when(sid < NS - 1)
            def _stage_live():
                for rr in range(2):
                    pltpu.sync_copy(eidx_hbm.at[rr].at[pl.ds(eb, HALF * CHUNK)],
                                    idxs.at[rr].at[sid])

            @pl.when(sid == NS - 1)
            def _stage_mixed():
                for rr in range(2):
                    pltpu.sync_copy(
                        eidx_hbm.at[rr].at[pl.ds(eb, L15 * CHUNK)],
                        idxs.at[rr].at[sid].at[pl.ds(0, L15 * CHUNK)])
                    pltpu.sync_copy(
                        pad_hbm.at[rr],
                        idxs.at[rr].at[sid].at[pl.ds(L15 * CHUNK, PADC * CHUNK)])

        pltpu.sync_copy(idxs.at[0].at[sid].at[pl.ds(0, CHUNK)], srcv0)
        pltpu.sync_copy(idxs.at[1].at[sid].at[pl.ds(0, CHUNK)], dstv0)
        pltpu.async_copy(hsh.at[srcv0], rows_a, sem)

        def _pair(jj, carry):
            for s in range(2):
                j = 2 * jj + s
                p, q = s, 1 - s
                pltpu.make_async_copy(hsh.at[srcv[p]], rows[p], sem).wait()
                if s == 0:
                    nxt = True
                else:
                    nxt = jj < HALF // 2 - 1

                def _prefetch():
                    nb = pl.multiple_of((j + 1) * CHUNK, 8)
                    pltpu.sync_copy(idxs.at[0].at[sid].at[pl.ds(nb, CHUNK)],
                                    srcv[q])
                    pltpu.sync_copy(idxs.at[1].at[sid].at[pl.ds(nb, CHUNK)],
                                    dstv[q])
                    pltpu.async_copy(hsh.at[srcv[q]], rows[q], sem)

                if nxt is True:
                    _prefetch()
                else:
                    pl.when(nxt)(_prefetch)
                pltpu.sync_copy(rows[p], acc.at[dstv[p]], add=True)
            return carry

        lax.fori_loop(0, HALF // 2, _pair, 0)

    plsc.subcore_barrier()

    obase = pl.multiple_of(sid * OPT, 8)
    cob = pl.multiple_of(cid * DH, 8)
    for k in range(OPT // CHUNK):
        pltpu.sync_copy(acc.at[pl.ds(obase + k * CHUNK, CHUNK)],
                        out_hbm.at[pl.ds(obase + k * CHUNK, CHUNK), pl.ds(cob, DH)])
    orem = OPT % CHUNK
    if orem:
        pltpu.sync_copy(acc.at[pl.ds(obase + (OPT // CHUNK) * CHUNK, orem)],
                        out_hbm.at[pl.ds(obase + (OPT // CHUNK) * CHUNK, orem),
                                   pl.ds(cob, DH)])

    @pl.when(sid == NS - 1)
    def _tail():
        pltpu.sync_copy(acc.at[pl.ds(OPT * NS, OREM)],
                        out_hbm.at[pl.ds(OPT * NS, OREM), pl.ds(cob, DH)])


def kernel(x, edge_index, W, b_dense, bias):
    eidx = edge_index.astype(jnp.int32)
    padi = jnp.concatenate(
        [jnp.zeros((1, PADC * CHUNK), jnp.int32),
         jnp.full((1, PADC * CHUNK), N_NODES, jnp.int32)], axis=0)

    b2 = b_dense.reshape(NC, 1, DH)
    W2 = jnp.stack([W[:, :DH], W[:, DH:]], axis=0)
    hflat = pl.pallas_call(
        _mm_body,
        grid=(10, NC),
        in_specs=[
            pl.BlockSpec((N_NODES // 10, D), lambda i, c: (i, 0)),
            pl.BlockSpec((1, D, DH), lambda i, c: (c, 0, 0)),
            pl.BlockSpec((1, 1, DH), lambda i, c: (c, 0, 0)),
        ],
        out_specs=pl.BlockSpec((N_NODES // 10, DH), lambda i, c: (c * 10 + i, 0)),
        out_shape=jax.ShapeDtypeStruct((NC * N_NODES, DH), jnp.float32),
    )(x, W2, b2)

    sc_fn = pl.kernel(
        _sc_body,
        out_type=jax.ShapeDtypeStruct((N_NODES, D), jnp.float32),
        mesh=plsc.VectorSubcoreMesh(core_axis_name="c", subcore_axis_name="s"),
        compiler_params=pltpu.CompilerParams(use_tc_tiling_on_sc=False),
        scratch_types=[
            pltpu.VMEM((CHUNK,), jnp.int32),
            pltpu.VMEM((CHUNK,), jnp.int32),
            pltpu.VMEM((CHUNK,), jnp.int32),
            pltpu.VMEM((CHUNK,), jnp.int32),
            pltpu.VMEM((CHUNK, DH), jnp.float32),
            pltpu.VMEM((CHUNK, DH), jnp.float32),
            pltpu.VMEM((DH,), jnp.float32),
            pltpu.VMEM_SHARED((2, NS, HALF * CHUNK), jnp.int32),
            pltpu.VMEM_SHARED((N_NODES, DH), jnp.float32),
            pltpu.VMEM_SHARED((N_PAD, DH), jnp.float32),
            pltpu.SemaphoreType.DMA,
        ],
    )
    bias2 = bias.reshape(NC, DH)
    return sc_fn(hflat, eidx, padi, bias2)
